# baseline (speedup 1.0000x reference)
import functools

import jax
import jax.numpy as jnp
from jax import lax
from jax.experimental import pallas as pl
from jax.experimental.pallas import tpu as pltpu
from jax.experimental.pallas import tpu_sc as plsc

E = 160000
D = 128
S = 5
R = 2 * S + 2
C = 10
GROWS = R * C
NCH = E // C
NW = 32
CPW = NCH // NW
NBUF = 4
VPR = D // 16


def _make_sc_kernel():
    mesh = plsc.VectorSubcoreMesh(core_axis_name="c", subcore_axis_name="s")

    @functools.partial(
        pl.kernel,
        mesh=mesh,
        out_type=(
            jax.ShapeDtypeStruct((NCH, C, 2 * D), jnp.float32),
            jax.ShapeDtypeStruct((NCH, C, 2 * D), jnp.float32),
        ),
        scratch_types=(
            [
                pltpu.VMEM((NBUF, GROWS), jnp.int32),
                pltpu.VMEM((NBUF, GROWS, D), jnp.float32),
                pltpu.VMEM((NBUF, C, D), jnp.float32),
                pltpu.VMEM((NBUF, C, D), jnp.float32),
            ]
            + [pltpu.SemaphoreType.DMA] * (3 * NBUF)
        ),
    )
    def k(x_hbm, idx_hbm, ea_hbm, nb_hbm, idxv, buf, nbl, nbr, *sems):
        gsem = sems[0:NBUF]
        isem = sems[NBUF:2 * NBUF]
        osem = sems[2 * NBUF:3 * NBUF]
        wid = lax.axis_index("s") * 2 + lax.axis_index("c")
        cbase = wid * CPW

        def fire_idx(j, b):
            pltpu.async_copy(idx_hbm.at[cbase + j], idxv.at[b], isem[b])

        def wait_idx(b):
            pltpu.make_async_copy(idx_hbm.at[cbase], idxv.at[b],
                                  isem[b]).wait()

        def fire_gather(b):
            pltpu.async_copy(x_hbm.at[idxv.at[b]], buf.at[b], gsem[b])

        def wait_gather(b):
            pltpu.make_async_copy(x_hbm.at[idxv.at[b]], buf.at[b],
                                  gsem[b]).wait()

        def fire_out(j, b):
            ch = cbase + j
            pltpu.async_copy(buf.at[b, pl.ds(0, C)],
                             ea_hbm.at[ch, :, pl.ds(0, D)], osem[b])
            pltpu.async_copy(buf.at[b, pl.ds(C, C)],
                             ea_hbm.at[ch, :, pl.ds(D, D)], osem[b])
            pltpu.async_copy(nbl.at[b],
                             nb_hbm.at[ch, :, pl.ds(0, D)], osem[b])
            pltpu.async_copy(nbr.at[b],
                             nb_hbm.at[ch, :, pl.ds(D, D)], osem[b])

        def wait_out(b):
            for _ in range(4):
                pltpu.make_async_copy(
                    nbl.at[b], nb_hbm.at[0, :, pl.ds(0, D)],
                    osem[b]).wait()

        def compute(b):
            def cbody(c, cc):
                for v in range(VPR):
                    sl = pl.ds(16 * v, 16)
                    s_ = buf[b, c, sl]
                    d_ = buf[b, C + c, sl]
                    nbl[b, c, sl] = (s_ + d_) * 0.5
                    acc = buf[b, 2 * C + c, sl]
                    for r in range(3, R):
                        acc = acc + buf[b, r * C + c, sl]
                    nbr[b, c, sl] = acc * jnp.float32(1.0 / (2 * S))
                return cc

            lax.fori_loop(0, C, cbody, 0)

        def do_slot(j, b, with_owait, fire_next):
            b2 = (b + 2) % NBUF
            b3 = (b + 3) % NBUF
            wait_gather(b)
            compute(b)
            fire_out(j, b)
            if with_owait:
                wait_out(b2)
            if fire_next:
                wait_idx(b2)
                fire_gather(b2)
                fire_idx(jnp.minimum(j + 3, CPW - 1), b3)

        pltpu.sync_copy(idx_hbm.at[cbase + 0], idxv.at[0])
        pltpu.sync_copy(idx_hbm.at[cbase + 1], idxv.at[1])
        fire_gather(0)
        fire_gather(1)
        fire_idx(2, 2)

        do_slot(jnp.int32(0), 0, with_owait=False, fire_next=True)
        do_slot(jnp.int32(1), 1, with_owait=False, fire_next=True)

        def body(i, carry):
            jb = 4 * i + 2
            for u in range(4):
                do_slot(jb + u, (2 + u) % NBUF, with_owait=True,
                        fire_next=True)
            return carry

        lax.fori_loop(0, (CPW - 4) // 4, body, 0)

        do_slot(jnp.int32(CPW - 2), (CPW - 2) % NBUF, with_owait=True,
                fire_next=False)
        do_slot(jnp.int32(CPW - 1), (CPW - 1) % NBUF, with_owait=True,
                fire_next=False)

        wait_out((CPW - 2) % NBUF)
        wait_out((CPW - 1) % NBUF)
        wait_idx(CPW % NBUF)

    return k


_sc_agg = _make_sc_kernel()


def kernel(x, edge_index, nb_idx):
    src = edge_index[0]
    dst = edge_index[1]
    idx_full = jnp.concatenate(
        [src[None, :], dst[None, :],
         jnp.transpose(nb_idx[0]), jnp.transpose(nb_idx[1])],
        axis=0,
    )
    idx_ch = (
        idx_full.reshape(R, NCH, C)
        .transpose(1, 0, 2)
        .reshape(NCH, GROWS)
    )
    ea, nb = _sc_agg(x, idx_ch)
    return ea.reshape(E, 2 * D), nb.reshape(E, 2 * D)
